# Initial kernel scaffold; baseline (speedup 1.0000x reference)
import jax
import jax.numpy as jnp
from jax.experimental import pallas as pl


def kernel(x, edge_index, Wself0, Wneigh0, b0, Wself1, Wneigh1, b1, Wself2, Wneigh2, b2):
    raise NotImplementedError("write your pallas kernel here")


# speedup vs baseline: 3.1205x; 3.1205x over previous
import jax
import jax.numpy as jnp
from jax import lax
from jax.experimental import pallas as pl
from jax.experimental.pallas import tpu as pltpu
from jax.experimental.pallas import tpu_sc as plsc

N_CORES = 2
N_SUBCORES = 16
N_WORKERS = N_CORES * N_SUBCORES
CHUNK = 128
SUP = 8
ROWS_PER_TILE = 632
NP = N_SUBCORES * ROWS_PER_TILE
DEG_W = 128
BR = 2000


def _cdiv(a, b):
    return (a + b - 1) // b


def _sc_mesh():
    return plsc.VectorSubcoreMesh(
        core_axis_name="c", subcore_axis_name="s",
        num_cores=N_CORES, num_subcores=N_SUBCORES)


def _make_agg(chunks, dout):
    assert chunks % SUP == 0

    def body(src_hbm, dst_hbm, hn_hbm, zacc_hbm,
             acc_out, src_v, dst_v, rows_v, acc_sh):
        cid = lax.axis_index("c")
        sid = lax.axis_index("s")
        wid = cid * N_SUBCORES + sid
        r0 = sid * ROWS_PER_TILE
        pltpu.sync_copy(zacc_hbm.at[pl.ds(r0, ROWS_PER_TILE)],
                        acc_sh.at[pl.ds(r0, ROWS_PER_TILE)])
        plsc.subcore_barrier()

        def outer(k, carry):
            base = pl.multiple_of(k * SUP, SUP)
            pltpu.sync_copy(src_hbm.at[wid, pl.ds(base, SUP)], src_v)
            pltpu.sync_copy(dst_hbm.at[wid, pl.ds(base, SUP)], dst_v)
            for j in range(SUP):
                pltpu.sync_copy(hn_hbm.at[src_v.at[j]], rows_v)
                pltpu.sync_copy(rows_v, acc_sh.at[dst_v.at[j]], add=True)
            return carry

        lax.fori_loop(0, chunks // SUP, outer, 0)
        plsc.subcore_barrier()
        pltpu.sync_copy(acc_sh.at[pl.ds(r0, ROWS_PER_TILE)],
                        acc_out.at[cid, pl.ds(r0, ROWS_PER_TILE)])

    return pl.kernel(
        body,
        out_type=jax.ShapeDtypeStruct((N_CORES, NP, dout), jnp.float32),
        mesh=_sc_mesh(),
        scratch_types=[
            pltpu.VMEM((SUP, CHUNK), jnp.int32),
            pltpu.VMEM((SUP, CHUNK), jnp.int32),
            pltpu.VMEM((CHUNK, dout), jnp.float32),
            pltpu.VMEM_SHARED((NP, dout), jnp.float32),
        ])


def _make_deg(chunks):
    assert chunks % SUP == 0

    def body(dst_hbm, zdeg_hbm, ones_hbm,
             deg_out, dst_v, ones_v, deg_sh):
        cid = lax.axis_index("c")
        sid = lax.axis_index("s")
        wid = cid * N_SUBCORES + sid
        r0 = sid * ROWS_PER_TILE
        pltpu.sync_copy(zdeg_hbm.at[pl.ds(r0, ROWS_PER_TILE)],
                        deg_sh.at[pl.ds(r0, ROWS_PER_TILE)])
        pltpu.sync_copy(ones_hbm, ones_v)
        plsc.subcore_barrier()

        def outer(k, carry):
            base = pl.multiple_of(k * SUP, SUP)
            pltpu.sync_copy(dst_hbm.at[wid, pl.ds(base, SUP)], dst_v)
            for j in range(SUP):
                pltpu.sync_copy(ones_v, deg_sh.at[dst_v.at[j]], add=True)
            return carry

        lax.fori_loop(0, chunks // SUP, outer, 0)
        plsc.subcore_barrier()
        pltpu.sync_copy(deg_sh.at[pl.ds(r0, ROWS_PER_TILE)],
                        deg_out.at[cid, pl.ds(r0, ROWS_PER_TILE)])

    return pl.kernel(
        body,
        out_type=jax.ShapeDtypeStruct((N_CORES, NP, DEG_W), jnp.float32),
        mesh=_sc_mesh(),
        scratch_types=[
            pltpu.VMEM((SUP, CHUNK), jnp.int32),
            pltpu.VMEM((CHUNK, DEG_W), jnp.float32),
            pltpu.VMEM_SHARED((NP, DEG_W), jnp.float32),
        ])


def _tc_pre(x, Wself, Wneigh, b):
    n, din = x.shape
    dout = Wself.shape[1]
    grid = n // BR

    def body(x_ref, ws_ref, wn_ref, b_ref, hs_ref, hn_ref):
        xb = x_ref[...]
        hs_ref[...] = (jnp.dot(xb, ws_ref[...], preferred_element_type=jnp.float32)
                       + b_ref[...])
        hn_ref[...] = jnp.dot(xb, wn_ref[...], preferred_element_type=jnp.float32)

    return pl.pallas_call(
        body,
        grid=(grid,),
        in_specs=[
            pl.BlockSpec((BR, din), lambda i: (i, 0)),
            pl.BlockSpec((din, dout), lambda i: (0, 0)),
            pl.BlockSpec((din, dout), lambda i: (0, 0)),
            pl.BlockSpec((1, dout), lambda i: (0, 0)),
        ],
        out_specs=[
            pl.BlockSpec((BR, dout), lambda i: (i, 0)),
            pl.BlockSpec((BR, dout), lambda i: (i, 0)),
        ],
        out_shape=[
            jax.ShapeDtypeStruct((n, dout), jnp.float32),
            jax.ShapeDtypeStruct((n, dout), jnp.float32),
        ],
    )(x, Wself, Wneigh, b.reshape(1, dout))


def _tc_mid(hsp, accA, accB, degA, degB, Wself, Wneigh, b):
    n, din = hsp.shape
    dout_s = Wself.shape[1]
    dout_n = Wneigh.shape[1]
    grid = n // BR

    def body(hsp_ref, aA_ref, aB_ref, dA_ref, dB_ref, ws_ref, wn_ref, b_ref,
             hs_ref, hn_ref):
        deg = dA_ref[:, :1] + dB_ref[:, :1]
        mean = (aA_ref[...] + aB_ref[...]) / jnp.maximum(deg, 1.0)
        h = jnp.maximum(hsp_ref[...] + mean, 0.0)
        hs_ref[...] = (jnp.dot(h, ws_ref[...], preferred_element_type=jnp.float32)
                       + b_ref[...])
        hn_ref[...] = jnp.dot(h, wn_ref[...], preferred_element_type=jnp.float32)

    return pl.pallas_call(
        body,
        grid=(grid,),
        in_specs=[
            pl.BlockSpec((BR, din), lambda i: (i, 0)),
            pl.BlockSpec((BR, din), lambda i: (i, 0)),
            pl.BlockSpec((BR, din), lambda i: (i, 0)),
            pl.BlockSpec((BR, DEG_W), lambda i: (i, 0)),
            pl.BlockSpec((BR, DEG_W), lambda i: (i, 0)),
            pl.BlockSpec((din, dout_s), lambda i: (0, 0)),
            pl.BlockSpec((din, dout_n), lambda i: (0, 0)),
            pl.BlockSpec((1, dout_s), lambda i: (0, 0)),
        ],
        out_specs=[
            pl.BlockSpec((BR, dout_s), lambda i: (i, 0)),
            pl.BlockSpec((BR, dout_n), lambda i: (i, 0)),
        ],
        out_shape=[
            jax.ShapeDtypeStruct((n, dout_s), jnp.float32),
            jax.ShapeDtypeStruct((n, dout_n), jnp.float32),
        ],
    )(hsp, accA, accB, degA, degB, Wself, Wneigh, b.reshape(1, dout_s))


def _tc_post(hsp, accA, accB, degA, degB):
    n, dout = hsp.shape
    dacc = accA.shape[1]
    grid = n // BR

    def body(hsp_ref, aA_ref, aB_ref, dA_ref, dB_ref, out_ref):
        deg = dA_ref[:, :1] + dB_ref[:, :1]
        mean = (aA_ref[:, :dout] + aB_ref[:, :dout]) / jnp.maximum(deg, 1.0)
        out_ref[...] = hsp_ref[...] + mean

    return pl.pallas_call(
        body,
        grid=(grid,),
        in_specs=[
            pl.BlockSpec((BR, dout), lambda i: (i, 0)),
            pl.BlockSpec((BR, dacc), lambda i: (i, 0)),
            pl.BlockSpec((BR, dacc), lambda i: (i, 0)),
            pl.BlockSpec((BR, DEG_W), lambda i: (i, 0)),
            pl.BlockSpec((BR, DEG_W), lambda i: (i, 0)),
        ],
        out_specs=pl.BlockSpec((BR, dout), lambda i: (i, 0)),
        out_shape=jax.ShapeDtypeStruct((n, dout), jnp.float32),
    )(hsp, accA, accB, degA, degB)


def kernel(x, edge_index, Wself0, Wneigh0, b0, Wself1, Wneigh1, b1,
           Wself2, Wneigh2, b2):
    n = x.shape[0]
    e = edge_index.shape[1]
    chunks = SUP * _cdiv(e, N_WORKERS * CHUNK * SUP)
    e_pad = chunks * CHUNK * N_WORKERS

    src = edge_index[0].astype(jnp.int32)
    dst = edge_index[1].astype(jnp.int32)
    src_p = jnp.concatenate(
        [src, jnp.zeros((e_pad - e,), jnp.int32)]).reshape(N_WORKERS, chunks, CHUNK)
    dst_p = jnp.concatenate(
        [dst, jnp.full((e_pad - e,), n, jnp.int32)]).reshape(N_WORKERS, chunks, CHUNK)

    z128 = jnp.zeros((NP, 128), jnp.float32)
    zdeg = jnp.zeros((NP, DEG_W), jnp.float32)
    ones = jnp.ones((CHUNK, DEG_W), jnp.float32)

    Wneigh2p = jnp.concatenate(
        [Wneigh2, jnp.zeros((Wneigh2.shape[0], 128 - Wneigh2.shape[1]),
                            jnp.float32)], axis=1)

    agg128 = _make_agg(chunks, 128)
    deg_k = _make_deg(chunks)

    deg = deg_k(dst_p, zdeg, ones)
    hs0, hn0 = _tc_pre(x, Wself0, Wneigh0, b0)
    acc0 = agg128(src_p, dst_p, hn0, z128)
    hs1, hn1 = _tc_mid(hs0, acc0[0], acc0[1], deg[0], deg[1],
                       Wself1, Wneigh1, b1)
    acc1 = agg128(src_p, dst_p, hn1, z128)
    hs2, hn2 = _tc_mid(hs1, acc1[0], acc1[1], deg[0], deg[1],
                       Wself2, Wneigh2p, b2)
    acc2 = agg128(src_p, dst_p, hn2, z128)
    out = _tc_post(hs2, acc2[0], acc2[1], deg[0], deg[1])
    return out


# speedup vs baseline: 4.7693x; 1.5284x over previous
import jax
import jax.numpy as jnp
from jax import lax
from jax.experimental import pallas as pl
from jax.experimental.pallas import tpu as pltpu
from jax.experimental.pallas import tpu_sc as plsc

N_CORES = 2
N_SUBCORES = 16
N_WORKERS = N_CORES * N_SUBCORES
CHUNK = 64
ROWS_PER_TILE = 632
NP = N_SUBCORES * ROWS_PER_TILE
DEG_W = 128
BR = 2000


def _cdiv(a, b):
    return (a + b - 1) // b


def _edge_layout(e):
    chunks = 2 * _cdiv(e, N_WORKERS * CHUNK * 2)
    return chunks, chunks * CHUNK * N_WORKERS


def _sc_mesh():
    return plsc.VectorSubcoreMesh(
        core_axis_name="c", subcore_axis_name="s",
        num_cores=N_CORES, num_subcores=N_SUBCORES)


def _make_agg(chunks, dout):
    T = chunks
    assert T % 2 == 0 and T >= 4

    def body(src_hbm, dst_hbm, hn_hbm, zacc_hbm, acc_out,
             src_v, dst_v, rows0, rows1, acc_sh, gs0, gs1, ss0, ss1):
        cid = lax.axis_index("c")
        sid = lax.axis_index("s")
        wid = cid * N_SUBCORES + sid
        r0 = sid * ROWS_PER_TILE
        pltpu.sync_copy(zacc_hbm.at[pl.ds(r0, ROWS_PER_TILE)],
                        acc_sh.at[pl.ds(r0, ROWS_PER_TILE)])
        pltpu.sync_copy(src_hbm.at[wid], src_v)
        pltpu.sync_copy(dst_hbm.at[wid], dst_v)
        plsc.subcore_barrier()

        rows = (rows0, rows1)
        gsem = (gs0, gs1)
        ssem = (ss0, ss1)

        def g(c, b):
            base = pl.multiple_of(c * CHUNK, CHUNK)
            pltpu.async_copy(hn_hbm.at[src_v.at[pl.ds(base, CHUNK)]],
                             rows[b], gsem[b])

        def gwait(b):
            pltpu.make_async_copy(hn_hbm.at[pl.ds(0, CHUNK)], rows[b],
                                  gsem[b]).wait()

        def s(c, b):
            pltpu.async_copy(rows[b], acc_sh.at[dst_v.at[c]], ssem[b],
                             add=True)

        def swait(b):
            pltpu.make_async_copy(rows[b], acc_sh.at[pl.ds(0, CHUNK)],
                                  ssem[b]).wait()

        g(0, 0)
        gwait(0)
        s(0, 0)
        g(1, 1)

        def step(t, carry):
            c = 2 * t
            gwait(1)
            s(c + 1, 1)
            swait(0)
            g(c + 2, 0)
            gwait(0)
            s(c + 2, 0)
            swait(1)
            g(c + 3, 1)
            return carry

        lax.fori_loop(0, T // 2 - 1, step, 0)
        gwait(1)
        s(T - 1, 1)
        swait(0)
        swait(1)
        plsc.subcore_barrier()
        pltpu.sync_copy(acc_sh.at[pl.ds(r0, ROWS_PER_TILE)],
                        acc_out.at[cid, pl.ds(r0, ROWS_PER_TILE)])

    return pl.kernel(
        body,
        out_type=jax.ShapeDtypeStruct((N_CORES, NP, dout), jnp.float32),
        mesh=_sc_mesh(),
        scratch_types=[
            pltpu.VMEM((T * CHUNK,), jnp.int32),
            pltpu.VMEM((T, CHUNK), jnp.int32),
            pltpu.VMEM((CHUNK, dout), jnp.float32),
            pltpu.VMEM((CHUNK, dout), jnp.float32),
            pltpu.VMEM_SHARED((NP, dout), jnp.float32),
            pltpu.SemaphoreType.DMA,
            pltpu.SemaphoreType.DMA,
            pltpu.SemaphoreType.DMA,
            pltpu.SemaphoreType.DMA,
        ])


def _make_deg(chunks):
    T = chunks

    def body(dst_hbm, zdeg_hbm, ones_hbm,
             deg_out, dst_v, ones_v, deg_sh):
        cid = lax.axis_index("c")
        sid = lax.axis_index("s")
        wid = cid * N_SUBCORES + sid
        r0 = sid * ROWS_PER_TILE
        pltpu.sync_copy(zdeg_hbm.at[pl.ds(r0, ROWS_PER_TILE)],
                        deg_sh.at[pl.ds(r0, ROWS_PER_TILE)])
        pltpu.sync_copy(ones_hbm, ones_v)
        pltpu.sync_copy(dst_hbm.at[wid], dst_v)
        plsc.subcore_barrier()

        def step(j, carry):
            pltpu.sync_copy(ones_v, deg_sh.at[dst_v.at[j]], add=True)
            return carry

        lax.fori_loop(0, T, step, 0)
        plsc.subcore_barrier()
        pltpu.sync_copy(deg_sh.at[pl.ds(r0, ROWS_PER_TILE)],
                        deg_out.at[cid, pl.ds(r0, ROWS_PER_TILE)])

    return pl.kernel(
        body,
        out_type=jax.ShapeDtypeStruct((N_CORES, NP, DEG_W), jnp.float32),
        mesh=_sc_mesh(),
        scratch_types=[
            pltpu.VMEM((T, CHUNK), jnp.int32),
            pltpu.VMEM((CHUNK, DEG_W), jnp.float32),
            pltpu.VMEM_SHARED((NP, DEG_W), jnp.float32),
        ])


def _tc_pre(x, Wself, Wneigh, b):
    n, din = x.shape
    dout = Wself.shape[1]
    grid = n // BR

    def body(x_ref, ws_ref, wn_ref, b_ref, hs_ref, hn_ref):
        xb = x_ref[...]
        hs_ref[...] = (jnp.dot(xb, ws_ref[...], preferred_element_type=jnp.float32)
                       + b_ref[...])
        hn_ref[...] = jnp.dot(xb, wn_ref[...], preferred_element_type=jnp.float32)

    return pl.pallas_call(
        body,
        grid=(grid,),
        in_specs=[
            pl.BlockSpec((BR, din), lambda i: (i, 0)),
            pl.BlockSpec((din, dout), lambda i: (0, 0)),
            pl.BlockSpec((din, dout), lambda i: (0, 0)),
            pl.BlockSpec((1, dout), lambda i: (0, 0)),
        ],
        out_specs=[
            pl.BlockSpec((BR, dout), lambda i: (i, 0)),
            pl.BlockSpec((BR, dout), lambda i: (i, 0)),
        ],
        out_shape=[
            jax.ShapeDtypeStruct((n, dout), jnp.float32),
            jax.ShapeDtypeStruct((n, dout), jnp.float32),
        ],
    )(x, Wself, Wneigh, b.reshape(1, dout))


def _tc_mid(hsp, accA, accB, degA, degB, Wself, Wneigh, b):
    n, din = hsp.shape
    dout_s = Wself.shape[1]
    dout_n = Wneigh.shape[1]
    grid = n // BR

    def body(hsp_ref, aA_ref, aB_ref, dA_ref, dB_ref, ws_ref, wn_ref, b_ref,
             hs_ref, hn_ref):
        deg = dA_ref[:, :1] + dB_ref[:, :1]
        mean = (aA_ref[...] + aB_ref[...]) / jnp.maximum(deg, 1.0)
        h = jnp.maximum(hsp_ref[...] + mean, 0.0)
        hs_ref[...] = (jnp.dot(h, ws_ref[...], preferred_element_type=jnp.float32)
                       + b_ref[...])
        hn_ref[...] = jnp.dot(h, wn_ref[...], preferred_element_type=jnp.float32)

    return pl.pallas_call(
        body,
        grid=(grid,),
        in_specs=[
            pl.BlockSpec((BR, din), lambda i: (i, 0)),
            pl.BlockSpec((BR, din), lambda i: (i, 0)),
            pl.BlockSpec((BR, din), lambda i: (i, 0)),
            pl.BlockSpec((BR, DEG_W), lambda i: (i, 0)),
            pl.BlockSpec((BR, DEG_W), lambda i: (i, 0)),
            pl.BlockSpec((din, dout_s), lambda i: (0, 0)),
            pl.BlockSpec((din, dout_n), lambda i: (0, 0)),
            pl.BlockSpec((1, dout_s), lambda i: (0, 0)),
        ],
        out_specs=[
            pl.BlockSpec((BR, dout_s), lambda i: (i, 0)),
            pl.BlockSpec((BR, dout_n), lambda i: (i, 0)),
        ],
        out_shape=[
            jax.ShapeDtypeStruct((n, dout_s), jnp.float32),
            jax.ShapeDtypeStruct((n, dout_n), jnp.float32),
        ],
    )(hsp, accA, accB, degA, degB, Wself, Wneigh, b.reshape(1, dout_s))


def _tc_post(hsp, accA, accB, degA, degB):
    n, dout = hsp.shape
    dacc = accA.shape[1]
    grid = n // BR

    def body(hsp_ref, aA_ref, aB_ref, dA_ref, dB_ref, out_ref):
        deg = dA_ref[:, :1] + dB_ref[:, :1]
        mean = (aA_ref[:, :dout] + aB_ref[:, :dout]) / jnp.maximum(deg, 1.0)
        out_ref[...] = hsp_ref[...] + mean

    return pl.pallas_call(
        body,
        grid=(grid,),
        in_specs=[
            pl.BlockSpec((BR, dout), lambda i: (i, 0)),
            pl.BlockSpec((BR, dacc), lambda i: (i, 0)),
            pl.BlockSpec((BR, dacc), lambda i: (i, 0)),
            pl.BlockSpec((BR, DEG_W), lambda i: (i, 0)),
            pl.BlockSpec((BR, DEG_W), lambda i: (i, 0)),
        ],
        out_specs=pl.BlockSpec((BR, dout), lambda i: (i, 0)),
        out_shape=jax.ShapeDtypeStruct((n, dout), jnp.float32),
    )(hsp, accA, accB, degA, degB)


def kernel(x, edge_index, Wself0, Wneigh0, b0, Wself1, Wneigh1, b1,
           Wself2, Wneigh2, b2):
    n = x.shape[0]
    e = edge_index.shape[1]
    chunks, e_pad = _edge_layout(e)

    src = edge_index[0].astype(jnp.int32)
    dst = edge_index[1].astype(jnp.int32)
    src_p = jnp.concatenate(
        [src, jnp.zeros((e_pad - e,), jnp.int32)]).reshape(N_WORKERS, chunks * CHUNK)
    dst_p = jnp.concatenate(
        [dst, jnp.full((e_pad - e,), n, jnp.int32)]).reshape(N_WORKERS, chunks, CHUNK)

    z128 = jnp.zeros((NP, 128), jnp.float32)
    zdeg = jnp.zeros((NP, DEG_W), jnp.float32)
    ones = jnp.ones((CHUNK, DEG_W), jnp.float32)

    Wneigh2p = jnp.concatenate(
        [Wneigh2, jnp.zeros((Wneigh2.shape[0], 128 - Wneigh2.shape[1]),
                            jnp.float32)], axis=1)

    agg128 = _make_agg(chunks, 128)
    deg_k = _make_deg(chunks)

    deg = deg_k(dst_p, zdeg, ones)
    hs0, hn0 = _tc_pre(x, Wself0, Wneigh0, b0)
    acc0 = agg128(src_p, dst_p, hn0, z128)
    hs1, hn1 = _tc_mid(hs0, acc0[0], acc0[1], deg[0], deg[1],
                       Wself1, Wneigh1, b1)
    acc1 = agg128(src_p, dst_p, hn1, z128)
    hs2, hn2 = _tc_mid(hs1, acc1[0], acc1[1], deg[0], deg[1],
                       Wself2, Wneigh2p, b2)
    acc2 = agg128(src_p, dst_p, hn2, z128)
    out = _tc_post(hs2, acc2[0], acc2[1], deg[0], deg[1])
    return out
